# baseline (speedup 1.0000x reference)
import jax
import jax.numpy as jnp
from jax import lax
from jax.experimental import pallas as pl
from jax.experimental.pallas import tpu as pltpu
from jax.experimental.pallas import tpu_sc as plsc

N_SRC = 100000
N_DST = 100000
D = 128
E = 600000

NC = 2
NS = 16
NW = NC * NS
L = 16
NB = 3
STRIDE = L + 1

CHUNK = 64
N_CHUNKS = -(-E // (NW * CHUNK))
N_CHUNKS += (-N_CHUNKS) % NB
PER_W = N_CHUNKS * CHUNK
EP = NW * PER_W


def _body(xs_hbm, xd_hbm, i0_hbm, i1_hbm, out_hbm,
          i0_v, i1_v, rs0_v, rs1_v, rs2_v, rd0_v, rd1_v, rd2_v,
          o0_v, o1_v, o2_v, p_v,
          sem_rs, sem_rd, sem_out):
    cid = lax.axis_index("c")
    sid = lax.axis_index("s")
    wid = sid * NC + cid
    wbase = wid * PER_W

    rs_bufs = [rs0_v, rs1_v, rs2_v]
    rd_bufs = [rd0_v, rd1_v, rd2_v]
    o_bufs = [o0_v, o1_v, o2_v]

    pltpu.sync_copy(i0_hbm.at[pl.ds(wbase, PER_W)], i0_v)
    pltpu.sync_copy(i1_hbm.at[pl.ds(wbase, PER_W)], i1_v)

    def issue_rows(k, b):
        idx0 = i0_v.at[pl.ds(k * CHUNK, CHUNK)]
        idx1 = i1_v.at[pl.ds(k * CHUNK, CHUNK)]
        pltpu.async_copy(xs_hbm.at[idx0], rs_bufs[b], sem_rs[b])
        pltpu.async_copy(xd_hbm.at[idx1], rd_bufs[b], sem_rd[b])

    for b in range(NB):
        issue_rows(b, b)

    lanes = lax.iota(jnp.int32, L)
    col0 = lanes * STRIDE

    def super_body(ss, carry):
        for b in range(NB):
            k = ss * NB + b
            rs = rs_bufs[b]
            rd = rd_bufs[b]
            ob = o_bufs[b]
            pltpu.make_async_copy(xs_hbm.at[i0_v.at[pl.ds(0, CHUNK)]],
                                  rs, sem_rs[b]).wait()
            pltpu.make_async_copy(xd_hbm.at[i1_v.at[pl.ds(0, CHUNK)]],
                                  rd, sem_rd[b]).wait()

            @pl.when(k >= NB)
            def _():
                pltpu.make_async_copy(
                    ob, out_hbm.at[pl.ds(wbase, CHUNK)], sem_out[b]).wait()

            def group_body(g, gcarry):
                e0 = g * L
                for u in range(L):
                    e = e0 + u
                    acc0 = rs[e, pl.ds(0, L)] * rd[e, pl.ds(0, L)]
                    acc1 = rs[e, pl.ds(L, L)] * rd[e, pl.ds(L, L)]
                    for kk in range(2, D // L, 2):
                        acc0 = acc0 + rs[e, pl.ds(kk * L, L)] * rd[e, pl.ds(kk * L, L)]
                        acc1 = acc1 + rs[e, pl.ds((kk + 1) * L, L)] * rd[e, pl.ds((kk + 1) * L, L)]
                    plsc.store_scatter(p_v, [lanes + (STRIDE * u)],
                                       acc0 + acc1)
                r0 = plsc.load_gather(p_v, [col0])
                r1 = plsc.load_gather(p_v, [col0 + 1])
                for j in range(2, L, 2):
                    r0 = r0 + plsc.load_gather(p_v, [col0 + j])
                    r1 = r1 + plsc.load_gather(p_v, [col0 + (j + 1)])
                ob[pl.ds(e0, L)] = r0 + r1
                return gcarry

            lax.fori_loop(0, CHUNK // L, group_body, 0)

            pltpu.async_copy(
                ob, out_hbm.at[pl.ds(wbase + k * CHUNK, CHUNK)], sem_out[b])

            @pl.when(k + NB < N_CHUNKS)
            def _():
                issue_rows(k + NB, b)

        return carry

    lax.fori_loop(0, N_CHUNKS // NB, super_body, 0)

    for b in range(NB):
        pltpu.make_async_copy(
            o_bufs[b], out_hbm.at[pl.ds(wbase, CHUNK)], sem_out[b]).wait()


@jax.jit
def _run(x_src, x_dst, i0, i1):
    mesh = plsc.VectorSubcoreMesh(core_axis_name="c", subcore_axis_name="s")
    f = pl.kernel(
        _body,
        out_type=jax.ShapeDtypeStruct((EP,), jnp.float32),
        mesh=mesh,
        scratch_types=[
            pltpu.VMEM((PER_W,), jnp.int32),
            pltpu.VMEM((PER_W,), jnp.int32),
            pltpu.VMEM((CHUNK, D), jnp.float32),
            pltpu.VMEM((CHUNK, D), jnp.float32),
            pltpu.VMEM((CHUNK, D), jnp.float32),
            pltpu.VMEM((CHUNK, D), jnp.float32),
            pltpu.VMEM((CHUNK, D), jnp.float32),
            pltpu.VMEM((CHUNK, D), jnp.float32),
            pltpu.VMEM((CHUNK,), jnp.float32),
            pltpu.VMEM((CHUNK,), jnp.float32),
            pltpu.VMEM((CHUNK,), jnp.float32),
            pltpu.VMEM((L * STRIDE,), jnp.float32),
            [pltpu.SemaphoreType.DMA] * NB,
            [pltpu.SemaphoreType.DMA] * NB,
            [pltpu.SemaphoreType.DMA] * NB,
        ],
        compiler_params=pltpu.CompilerParams(needs_layout_passes=False),
    )
    return f(x_src, x_dst, i0, i1)


def kernel(x_src, x_dst, edge_label_index):
    tail = jnp.arange(EP - E, dtype=jnp.int32) % min(N_SRC, N_DST)
    i0 = jnp.concatenate([edge_label_index[0], tail])
    i1 = jnp.concatenate([edge_label_index[1], tail])
    out = _run(x_src, x_dst, i0, i1)
    return out[:E]
